# baseline (speedup 1.0000x reference)
import functools

import jax
import jax.numpy as jnp
from jax import lax
from jax.experimental import pallas as pl
from jax.experimental.pallas import tpu as pltpu
from jax.experimental.pallas import tpu_sc as plsc

BATCH = 4096
SEQ_LEN = 200
EMBED_DIM = 64

_info = plsc.get_sparse_core_info()
NC, NS = _info.num_cores, _info.num_subcores
NW = NC * NS
B_PER_W = BATCH // NW


def _gather_kernel(table_hbm, idx_hbm, out_hbm, idx_v, rows_v, gs0, gs1, os0, os1):
    gsem = (gs0, gs1)
    osem = (os0, os1)
    wid = lax.axis_index("s") * NC + lax.axis_index("c")
    bbase = wid * B_PER_W
    pltpu.sync_copy(idx_hbm.at[pl.ds(bbase, B_PER_W)], idx_v)

    def gather_desc(b, k):
        return pltpu.make_async_copy(
            table_hbm.at[idx_v.at[b]], rows_v.at[k, 0], gsem[k]
        )

    def oc_desc(b, k):
        return pltpu.make_async_copy(
            rows_v.at[k], out_hbm.at[pl.ds(bbase + b, 1)], osem[k]
        )

    gather_desc(0, 0).start()
    gather_desc(0, 0).wait()
    oc_desc(0, 0).start()
    gather_desc(1, 1).start()

    def body(t, _):
        b = 2 * t + 1
        gather_desc(b, 1).wait()
        oc_desc(b, 1).start()
        oc_desc(b - 1, 0).wait()
        gather_desc(b + 1, 0).start()

        b2 = b + 1
        gather_desc(b2, 0).wait()
        oc_desc(b2, 0).start()
        oc_desc(b2 - 1, 1).wait()
        gather_desc(b2 + 1, 1).start()
        return ()

    lax.fori_loop(0, (B_PER_W - 2) // 2, body, ())

    bl = B_PER_W - 1
    gather_desc(bl, 1).wait()
    oc_desc(bl, 1).start()
    oc_desc(bl - 1, 0).wait()
    oc_desc(bl, 1).wait()


@jax.jit
def _run(table, indices):
    mesh = plsc.VectorSubcoreMesh(core_axis_name="c", subcore_axis_name="s")
    fn = functools.partial(
        pl.kernel,
        mesh=mesh,
        out_type=jax.ShapeDtypeStruct((BATCH, SEQ_LEN, EMBED_DIM), jnp.float32),
        scratch_types=[
            pltpu.VMEM((B_PER_W, SEQ_LEN), jnp.int32),
            pltpu.VMEM((2, 1, SEQ_LEN, EMBED_DIM), jnp.float32),
            pltpu.SemaphoreType.DMA,
            pltpu.SemaphoreType.DMA,
            pltpu.SemaphoreType.DMA,
            pltpu.SemaphoreType.DMA,
        ],
        compiler_params=pltpu.CompilerParams(use_tc_tiling_on_sc=False),
    )(_gather_kernel)
    return fn(table, indices)


def kernel(indices, table):
    return _run(table, indices.astype(jnp.int32))
